# Initial kernel scaffold; baseline (speedup 1.0000x reference)
import functools

import jax
import jax.numpy as jnp
from jax import lax
from jax.experimental import pallas as pl
from jax.experimental.pallas import tpu as pltpu
from jax.experimental.pallas import tpu_sc as plsc

N = 10000
M = 320000
D = 32
DA = 64
NC = 2
NS = 16
NW = NC * NS
PER_TILE = M // NW
CH = 128
NCH = -(-PER_TILE // CH)
PT = NCH * CH
PADN = PT - PER_TILE
TRASH = 240
ROWS = N + TRASH
RPT = N // NS
RB = 2000
GRID = N // RB

_mesh = plsc.VectorSubcoreMesh(core_axis_name="c", subcore_axis_name="s")


@functools.partial(
    pl.kernel,
    out_type=(
        jax.ShapeDtypeStruct((NC, N, D), jnp.float32),
        jax.ShapeDtypeStruct((NC, N, 16), jnp.float32),
    ),
    mesh=_mesh,
    scratch_types=[
        pltpu.VMEM((NCH, CH), jnp.int32),
        pltpu.VMEM((NCH, CH), jnp.int32),
        pltpu.VMEM((CH, D), jnp.float32),
        pltpu.VMEM((CH, 16), jnp.float32),
        pltpu.VMEM((CH, D), jnp.float32),
        pltpu.VMEM((CH, 16), jnp.float32),
        pltpu.VMEM_SHARED((N, D), jnp.float32),
        pltpu.VMEM_SHARED((ROWS, D), jnp.float32),
        pltpu.VMEM_SHARED((ROWS, 16), jnp.float32),
    ],
)
def _sc_scatter_stage(src_hbm, gidx_hbm, sidx_hbm, acc_out, cnt_out,
                      gidx_v, sidx_v, rows_v, ones_v, zrow_v, zcnt_v,
                      src_sp, acc_sp, cnt_sp):
    c = lax.axis_index("c")
    s = lax.axis_index("s")
    wid = c * NS + s

    zf = jnp.zeros((16,), jnp.float32)
    of = jnp.ones((16,), jnp.float32)

    @pl.loop(0, CH)
    def _(r):
        zrow_v[r, pl.ds(0, 16)] = zf
        zrow_v[r, pl.ds(16, 16)] = zf
        zcnt_v[r, pl.ds(0, 16)] = zf
        ones_v[r, pl.ds(0, 16)] = of

    @pl.loop(0, ROWS // NS // CH)
    def _(k):
        base = s * (ROWS // NS) + k * CH
        pltpu.sync_copy(zrow_v, acc_sp.at[pl.ds(base, CH)])
        pltpu.sync_copy(zcnt_v, cnt_sp.at[pl.ds(base, CH)])

    pltpu.sync_copy(src_hbm.at[pl.ds(s * RPT, RPT)],
                    src_sp.at[pl.ds(s * RPT, RPT)])
    pltpu.sync_copy(gidx_hbm.at[wid], gidx_v)
    pltpu.sync_copy(sidx_hbm.at[wid], sidx_v)
    plsc.subcore_barrier()

    @pl.loop(0, NCH)
    def _(j):
        pltpu.sync_copy(src_sp.at[gidx_v.at[j]], rows_v)
        pltpu.sync_copy(rows_v, acc_sp.at[sidx_v.at[j]], add=True)
        pltpu.sync_copy(ones_v, cnt_sp.at[sidx_v.at[j]], add=True)

    plsc.subcore_barrier()
    pltpu.sync_copy(acc_sp.at[pl.ds(s * RPT, RPT)],
                    acc_out.at[c, pl.ds(s * RPT, RPT)])
    pltpu.sync_copy(cnt_sp.at[pl.ds(s * RPT, RPT)],
                    cnt_out.at[c, pl.ds(s * RPT, RPT)])


def _dot(a, b):
    return lax.dot_general(a, b, (((1,), (0,)), ((), ())),
                           preferred_element_type=jnp.float32,
                           precision=lax.Precision.HIGHEST)


def _xform_body(x_ref, w_ref, o_ref):
    o_ref[...] = _dot(x_ref[...], w_ref[...])


def _mid_body(p_ref, c_ref, o_ref):
    psum = p_ref[0] + p_ref[1]
    cnt = c_ref[0, :, 0:1] + c_ref[1, :, 0:1]
    inv = jnp.where(cnt > 0.0, 1.0 / cnt, 0.0)
    o_ref[...] = psum * inv


def _fin_body(p_ref, c_ref, bhg_ref, w1_ref, wfc_ref, bfc_ref, o_ref):
    psum = p_ref[0] + p_ref[1]
    cnt = c_ref[0, :, 0:1] + c_ref[1, :, 0:1]
    inv = jnp.where(cnt > 0.0, 1.0 / cnt, 0.0)
    h = jnp.maximum(psum * inv + bhg_ref[...], 0.0)
    h2 = jnp.maximum(_dot(h, w1_ref[...]), 0.0)
    o_ref[...] = _dot(h2, wfc_ref[...]) + bfc_ref[0, 0]


def kernel(x, hyperedge_index, W_hg, b_hg, W1, a1, a2, W_fc, b_fc):
    ni = hyperedge_index[0].reshape(NW, PER_TILE)
    ei = hyperedge_index[1].reshape(NW, PER_TILE)
    pad = jnp.arange(NW * PADN, dtype=jnp.int32).reshape(NW, PADN)
    gpad = pad % N
    spad = N + pad % TRASH
    gA = jnp.concatenate([ni, gpad], axis=1).reshape(NW, NCH, CH)
    sA = jnp.concatenate([ei, spad], axis=1).reshape(NW, NCH, CH)
    gB = jnp.concatenate([ei, gpad], axis=1).reshape(NW, NCH, CH)
    sB = jnp.concatenate([ni, spad], axis=1).reshape(NW, NCH, CH)

    xt = pl.pallas_call(
        _xform_body,
        grid=(GRID,),
        in_specs=[pl.BlockSpec((RB, 128), lambda i: (i, 0)),
                  pl.BlockSpec((128, D), lambda i: (0, 0))],
        out_specs=pl.BlockSpec((RB, D), lambda i: (i, 0)),
        out_shape=jax.ShapeDtypeStruct((N, D), jnp.float32),
    )(x, W_hg)

    pe, ce = _sc_scatter_stage(xt, gA, sA)

    out_e = pl.pallas_call(
        _mid_body,
        grid=(GRID,),
        in_specs=[pl.BlockSpec((NC, RB, D), lambda i: (0, i, 0)),
                  pl.BlockSpec((NC, RB, 16), lambda i: (0, i, 0))],
        out_specs=pl.BlockSpec((RB, D), lambda i: (i, 0)),
        out_shape=jax.ShapeDtypeStruct((N, D), jnp.float32),
    )(pe, ce)

    pn, cn = _sc_scatter_stage(out_e, gB, sB)

    y = pl.pallas_call(
        _fin_body,
        grid=(GRID,),
        in_specs=[pl.BlockSpec((NC, RB, D), lambda i: (0, i, 0)),
                  pl.BlockSpec((NC, RB, 16), lambda i: (0, i, 0)),
                  pl.BlockSpec((1, D), lambda i: (0, 0)),
                  pl.BlockSpec((D, DA), lambda i: (0, 0)),
                  pl.BlockSpec((DA, 1), lambda i: (0, 0)),
                  pl.BlockSpec((1, 1), lambda i: (0, 0))],
        out_specs=pl.BlockSpec((RB, 1), lambda i: (i, 0)),
        out_shape=jax.ShapeDtypeStruct((N, 1), jnp.float32),
    )(pn, cn, b_hg.reshape(1, D), W1, W_fc.reshape(DA, 1),
      b_fc.reshape(1, 1))
    return y


# speedup vs baseline: 33.9680x; 33.9680x over previous
import functools

import jax
import jax.numpy as jnp
from jax import lax
from jax.experimental import pallas as pl
from jax.experimental.pallas import tpu as pltpu
from jax.experimental.pallas import tpu_sc as plsc

N = 10000
M = 320000
D = 32
DA = 64
NC = 2
NS = 16
NW = NC * NS
PER_TILE = M // NW
CH = 128
NCH = -(-PER_TILE // CH)
PT = NCH * CH
PADN = PT - PER_TILE
TRASH = 240
ROWS = N + TRASH
RPT = ROWS // NS
RB = 2000
GRID = N // RB

_mesh = plsc.VectorSubcoreMesh(core_axis_name="c", subcore_axis_name="s")


@functools.partial(
    pl.kernel,
    out_type=(
        jax.ShapeDtypeStruct((NC, N, D), jnp.float32),
        jax.ShapeDtypeStruct((NC, N, 16), jnp.float32),
    ),
    mesh=_mesh,
    scratch_types=[
        pltpu.VMEM((NCH, CH), jnp.int32),
        pltpu.VMEM((NCH, CH), jnp.int32),
        pltpu.VMEM((NCH, CH), jnp.int32),
        pltpu.VMEM((CH, D), jnp.float32),
        pltpu.VMEM((CH, 16), jnp.float32),
        pltpu.VMEM((CH, 16), jnp.float32),
        pltpu.VMEM((CH, D), jnp.float32),
        pltpu.VMEM((CH, 16), jnp.float32),
        pltpu.VMEM_SHARED((ROWS, D), jnp.float32),
        pltpu.VMEM_SHARED((ROWS, 16), jnp.float32),
    ],
    compiler_params=pltpu.CompilerParams(use_tc_tiling_on_sc=False),
)
def _sc_stage_a(src_hbm, gidx_hbm, sidx_hbm, cidx_hbm, acc_out, cnt_out,
                gidx_v, sidx_v, cidx_v, rows_v, eones_v, nones_v,
                zrow_v, zcnt_v, acc_sp, cnt_sp):
    c = lax.axis_index("c")
    s = lax.axis_index("s")
    wid = c * NS + s

    zf = jnp.zeros((16,), jnp.float32)
    lane = lax.iota(jnp.int32, 16)
    e_pat = jnp.where(lane < 8, 1.0, 0.0).astype(jnp.float32)
    n_pat = jnp.where(lane < 8, 0.0, 1.0).astype(jnp.float32)

    @pl.loop(0, CH)
    def _(r):
        zrow_v[r, pl.ds(0, 16)] = zf
        zrow_v[r, pl.ds(16, 16)] = zf
        zcnt_v[r, pl.ds(0, 16)] = zf
        eones_v[r, pl.ds(0, 16)] = e_pat
        nones_v[r, pl.ds(0, 16)] = n_pat

    @pl.loop(0, RPT // CH)
    def _(k):
        base = s * RPT + k * CH
        pltpu.sync_copy(zrow_v, acc_sp.at[pl.ds(base, CH)])
        pltpu.sync_copy(zcnt_v, cnt_sp.at[pl.ds(base, CH)])

    sbase = pl.multiple_of(jnp.minimum(s * RPT, N - RPT), 16)
    pltpu.sync_copy(gidx_hbm.at[wid], gidx_v)
    pltpu.sync_copy(sidx_hbm.at[wid], sidx_v)
    pltpu.sync_copy(cidx_hbm.at[wid], cidx_v)
    plsc.subcore_barrier()

    @pl.loop(0, NCH)
    def _(j):
        pltpu.sync_copy(src_hbm.at[gidx_v.at[j]], rows_v)
        pltpu.sync_copy(rows_v, acc_sp.at[sidx_v.at[j]], add=True)
        pltpu.sync_copy(eones_v, cnt_sp.at[sidx_v.at[j]], add=True)
        pltpu.sync_copy(nones_v, cnt_sp.at[cidx_v.at[j]], add=True)

    plsc.subcore_barrier()
    pltpu.sync_copy(acc_sp.at[pl.ds(sbase, RPT)],
                    acc_out.at[c, pl.ds(sbase, RPT)])
    pltpu.sync_copy(cnt_sp.at[pl.ds(sbase, RPT)],
                    cnt_out.at[c, pl.ds(sbase, RPT)])


@functools.partial(
    pl.kernel,
    out_type=jax.ShapeDtypeStruct((NC, N, D), jnp.float32),
    mesh=_mesh,
    scratch_types=[
        pltpu.VMEM((NCH, CH), jnp.int32),
        pltpu.VMEM((NCH, CH), jnp.int32),
        pltpu.VMEM((CH, D), jnp.float32),
        pltpu.VMEM((CH, D), jnp.float32),
        pltpu.VMEM_SHARED((ROWS, D), jnp.float32),
    ],
    compiler_params=pltpu.CompilerParams(use_tc_tiling_on_sc=False),
)
def _sc_stage_b(src_hbm, gidx_hbm, sidx_hbm, acc_out,
                gidx_v, sidx_v, rows_v, zrow_v, acc_sp):
    c = lax.axis_index("c")
    s = lax.axis_index("s")
    wid = c * NS + s

    zf = jnp.zeros((16,), jnp.float32)

    @pl.loop(0, CH)
    def _(r):
        zrow_v[r, pl.ds(0, 16)] = zf
        zrow_v[r, pl.ds(16, 16)] = zf

    @pl.loop(0, RPT // CH)
    def _(k):
        pltpu.sync_copy(zrow_v, acc_sp.at[pl.ds(s * RPT + k * CH, CH)])

    sbase = pl.multiple_of(jnp.minimum(s * RPT, N - RPT), 16)
    pltpu.sync_copy(gidx_hbm.at[wid], gidx_v)
    pltpu.sync_copy(sidx_hbm.at[wid], sidx_v)
    plsc.subcore_barrier()

    @pl.loop(0, NCH)
    def _(j):
        pltpu.sync_copy(src_hbm.at[gidx_v.at[j]], rows_v)
        pltpu.sync_copy(rows_v, acc_sp.at[sidx_v.at[j]], add=True)

    plsc.subcore_barrier()
    pltpu.sync_copy(acc_sp.at[pl.ds(sbase, RPT)],
                    acc_out.at[c, pl.ds(sbase, RPT)])


def _dot(a, b):
    return lax.dot_general(a, b, (((1,), (0,)), ((), ())),
                           preferred_element_type=jnp.float32,
                           precision=lax.Precision.HIGHEST)


def _xform_body(x_ref, w_ref, o_ref):
    o_ref[...] = _dot(x_ref[...], w_ref[...])


def _mid_body(p_ref, c_ref, o_ref):
    psum = p_ref[0] + p_ref[1]
    cnt = c_ref[0, :, 0:1] + c_ref[1, :, 0:1]
    inv = jnp.where(cnt > 0.0, 1.0 / cnt, 0.0)
    o_ref[...] = psum * inv


def _fin_body(p_ref, c_ref, bhg_ref, w1_ref, wfc_ref, bfc_ref, o_ref):
    psum = p_ref[0] + p_ref[1]
    cnt = c_ref[0, :, 8:9] + c_ref[1, :, 8:9]
    inv = jnp.where(cnt > 0.0, 1.0 / cnt, 0.0)
    h = jnp.maximum(psum * inv + bhg_ref[...], 0.0)
    h2 = jnp.maximum(_dot(h, w1_ref[...]), 0.0)
    o_ref[...] = _dot(h2, wfc_ref[...]) + bfc_ref[0, 0]


def kernel(x, hyperedge_index, W_hg, b_hg, W1, a1, a2, W_fc, b_fc):
    ni = hyperedge_index[0].reshape(NW, PER_TILE)
    ei = hyperedge_index[1].reshape(NW, PER_TILE)
    pad = jnp.arange(NW * PADN, dtype=jnp.int32).reshape(NW, PADN)
    gpad = pad % N
    spad = N + pad % TRASH
    gA = jnp.concatenate([ni, gpad], axis=1).reshape(NW, NCH, CH)
    sA = jnp.concatenate([ei, spad], axis=1).reshape(NW, NCH, CH)
    gB = jnp.concatenate([ei, gpad], axis=1).reshape(NW, NCH, CH)
    sB = jnp.concatenate([ni, spad], axis=1).reshape(NW, NCH, CH)

    xt = pl.pallas_call(
        _xform_body,
        grid=(GRID,),
        in_specs=[pl.BlockSpec((RB, 128), lambda i: (i, 0)),
                  pl.BlockSpec((128, D), lambda i: (0, 0))],
        out_specs=pl.BlockSpec((RB, D), lambda i: (i, 0)),
        out_shape=jax.ShapeDtypeStruct((N, D), jnp.float32),
    )(x, W_hg)

    pe, ce = _sc_stage_a(xt, gA, sA, sB)

    out_e = pl.pallas_call(
        _mid_body,
        grid=(GRID,),
        in_specs=[pl.BlockSpec((NC, RB, D), lambda i: (0, i, 0)),
                  pl.BlockSpec((NC, RB, 16), lambda i: (0, i, 0))],
        out_specs=pl.BlockSpec((RB, D), lambda i: (i, 0)),
        out_shape=jax.ShapeDtypeStruct((N, D), jnp.float32),
    )(pe, ce)

    pn = _sc_stage_b(out_e, gB, sB)

    y = pl.pallas_call(
        _fin_body,
        grid=(GRID,),
        in_specs=[pl.BlockSpec((NC, RB, D), lambda i: (0, i, 0)),
                  pl.BlockSpec((NC, RB, 16), lambda i: (0, i, 0)),
                  pl.BlockSpec((1, D), lambda i: (0, 0)),
                  pl.BlockSpec((D, DA), lambda i: (0, 0)),
                  pl.BlockSpec((DA, 1), lambda i: (0, 0)),
                  pl.BlockSpec((1, 1), lambda i: (0, 0))],
        out_specs=pl.BlockSpec((RB, 1), lambda i: (i, 0)),
        out_shape=jax.ShapeDtypeStruct((N, 1), jnp.float32),
    )(pn, ce, b_hg.reshape(1, D), W1, W_fc.reshape(DA, 1),
      b_fc.reshape(1, 1))
    return y


# speedup vs baseline: 48.2621x; 1.4208x over previous
import functools

import jax
import jax.numpy as jnp
from jax import lax
from jax.experimental import pallas as pl
from jax.experimental.pallas import tpu as pltpu
from jax.experimental.pallas import tpu_sc as plsc

N = 10000
M = 320000
D = 32
DA = 64
NC = 2
NS = 16
NW = NC * NS
PER_TILE = M // NW
CH = 128
NCH = -(-PER_TILE // CH)
PT = NCH * CH
PADN = PT - PER_TILE
TRASH = 240
ROWS = N + TRASH
RPT = ROWS // NS
RB = 2000
GRID = N // RB

_mesh = plsc.VectorSubcoreMesh(core_axis_name="c", subcore_axis_name="s")


@functools.partial(
    pl.kernel,
    out_type=(
        jax.ShapeDtypeStruct((NC, N, D), jnp.float32),
        jax.ShapeDtypeStruct((NC, N, 16), jnp.float32),
    ),
    mesh=_mesh,
    scratch_types=[
        pltpu.VMEM((NCH, CH), jnp.int32),
        pltpu.VMEM((NCH, CH), jnp.int32),
        pltpu.VMEM((NCH, CH), jnp.int32),
        pltpu.VMEM((CH, D), jnp.float32),
        pltpu.VMEM((CH, D), jnp.float32),
        pltpu.SemaphoreType.DMA,
        pltpu.SemaphoreType.DMA,
        pltpu.VMEM((CH, 16), jnp.float32),
        pltpu.VMEM((CH, 16), jnp.float32),
        pltpu.VMEM((CH, D), jnp.float32),
        pltpu.VMEM((CH, 16), jnp.float32),
        pltpu.VMEM_SHARED((ROWS, D), jnp.float32),
        pltpu.VMEM_SHARED((ROWS, 16), jnp.float32),
    ],
    compiler_params=pltpu.CompilerParams(use_tc_tiling_on_sc=False),
)
def _sc_stage_a(src_hbm, gidx_hbm, sidx_hbm, cidx_hbm, acc_out, cnt_out,
                gidx_v, sidx_v, cidx_v, rows0_v, rows1_v, gsem0, gsem1,
                eones_v, nones_v, zrow_v, zcnt_v, acc_sp, cnt_sp):
    c = lax.axis_index("c")
    s = lax.axis_index("s")
    wid = c * NS + s

    zf = jnp.zeros((16,), jnp.float32)
    lane = lax.iota(jnp.int32, 16)
    e_pat = jnp.where(lane < 8, 1.0, 0.0).astype(jnp.float32)
    n_pat = jnp.where(lane < 8, 0.0, 1.0).astype(jnp.float32)

    @pl.loop(0, CH)
    def _(r):
        zrow_v[r, pl.ds(0, 16)] = zf
        zrow_v[r, pl.ds(16, 16)] = zf
        zcnt_v[r, pl.ds(0, 16)] = zf
        eones_v[r, pl.ds(0, 16)] = e_pat
        nones_v[r, pl.ds(0, 16)] = n_pat

    @pl.loop(0, RPT // CH)
    def _(k):
        base = s * RPT + k * CH
        pltpu.sync_copy(zrow_v, acc_sp.at[pl.ds(base, CH)])
        pltpu.sync_copy(zcnt_v, cnt_sp.at[pl.ds(base, CH)])

    sbase = pl.multiple_of(jnp.minimum(s * RPT, N - RPT), 16)
    pltpu.sync_copy(gidx_hbm.at[wid], gidx_v)
    pltpu.sync_copy(sidx_hbm.at[wid], sidx_v)
    pltpu.sync_copy(cidx_hbm.at[wid], cidx_v)
    pltpu.async_copy(src_hbm.at[gidx_v.at[0]], rows0_v, gsem0)
    pltpu.async_copy(src_hbm.at[gidx_v.at[1]], rows1_v, gsem1)
    plsc.subcore_barrier()

    @pl.loop(0, NCH - 1, step=2)
    def _(j):
        pltpu.make_async_copy(src_hbm.at[gidx_v.at[j]], rows0_v, gsem0).wait()
        pltpu.sync_copy(rows0_v, acc_sp.at[sidx_v.at[j]], add=True)
        pltpu.sync_copy(eones_v, cnt_sp.at[sidx_v.at[j]], add=True)
        pltpu.sync_copy(nones_v, cnt_sp.at[cidx_v.at[j]], add=True)

        @pl.when(j + 2 < NCH)
        def _():
            pltpu.async_copy(src_hbm.at[gidx_v.at[j + 2]], rows0_v, gsem0)

        pltpu.make_async_copy(src_hbm.at[gidx_v.at[j + 1]], rows1_v,
                              gsem1).wait()
        pltpu.sync_copy(rows1_v, acc_sp.at[sidx_v.at[j + 1]], add=True)
        pltpu.sync_copy(eones_v, cnt_sp.at[sidx_v.at[j + 1]], add=True)
        pltpu.sync_copy(nones_v, cnt_sp.at[cidx_v.at[j + 1]], add=True)

        @pl.when(j + 3 < NCH)
        def _():
            pltpu.async_copy(src_hbm.at[gidx_v.at[j + 3]], rows1_v, gsem1)

    pltpu.make_async_copy(src_hbm.at[gidx_v.at[NCH - 1]], rows0_v,
                          gsem0).wait()
    pltpu.sync_copy(rows0_v, acc_sp.at[sidx_v.at[NCH - 1]], add=True)
    pltpu.sync_copy(eones_v, cnt_sp.at[sidx_v.at[NCH - 1]], add=True)
    pltpu.sync_copy(nones_v, cnt_sp.at[cidx_v.at[NCH - 1]], add=True)

    plsc.subcore_barrier()
    pltpu.sync_copy(acc_sp.at[pl.ds(sbase, RPT)],
                    acc_out.at[c, pl.ds(sbase, RPT)])
    pltpu.sync_copy(cnt_sp.at[pl.ds(sbase, RPT)],
                    cnt_out.at[c, pl.ds(sbase, RPT)])


@functools.partial(
    pl.kernel,
    out_type=jax.ShapeDtypeStruct((NC, N, D), jnp.float32),
    mesh=_mesh,
    scratch_types=[
        pltpu.VMEM((NCH, CH), jnp.int32),
        pltpu.VMEM((NCH, CH), jnp.int32),
        pltpu.VMEM((CH, D), jnp.float32),
        pltpu.VMEM((CH, D), jnp.float32),
        pltpu.SemaphoreType.DMA,
        pltpu.SemaphoreType.DMA,
        pltpu.VMEM((CH, D), jnp.float32),
        pltpu.VMEM_SHARED((ROWS, D), jnp.float32),
    ],
    compiler_params=pltpu.CompilerParams(use_tc_tiling_on_sc=False),
)
def _sc_stage_b(src_hbm, gidx_hbm, sidx_hbm, acc_out,
                gidx_v, sidx_v, rows0_v, rows1_v, gsem0, gsem1,
                zrow_v, acc_sp):
    c = lax.axis_index("c")
    s = lax.axis_index("s")
    wid = c * NS + s

    zf = jnp.zeros((16,), jnp.float32)

    @pl.loop(0, CH)
    def _(r):
        zrow_v[r, pl.ds(0, 16)] = zf
        zrow_v[r, pl.ds(16, 16)] = zf

    @pl.loop(0, RPT // CH)
    def _(k):
        pltpu.sync_copy(zrow_v, acc_sp.at[pl.ds(s * RPT + k * CH, CH)])

    sbase = pl.multiple_of(jnp.minimum(s * RPT, N - RPT), 16)
    pltpu.sync_copy(gidx_hbm.at[wid], gidx_v)
    pltpu.sync_copy(sidx_hbm.at[wid], sidx_v)
    pltpu.async_copy(src_hbm.at[gidx_v.at[0]], rows0_v, gsem0)
    pltpu.async_copy(src_hbm.at[gidx_v.at[1]], rows1_v, gsem1)
    plsc.subcore_barrier()

    @pl.loop(0, NCH - 1, step=2)
    def _(j):
        pltpu.make_async_copy(src_hbm.at[gidx_v.at[j]], rows0_v, gsem0).wait()
        pltpu.sync_copy(rows0_v, acc_sp.at[sidx_v.at[j]], add=True)

        @pl.when(j + 2 < NCH)
        def _():
            pltpu.async_copy(src_hbm.at[gidx_v.at[j + 2]], rows0_v, gsem0)

        pltpu.make_async_copy(src_hbm.at[gidx_v.at[j + 1]], rows1_v,
                              gsem1).wait()
        pltpu.sync_copy(rows1_v, acc_sp.at[sidx_v.at[j + 1]], add=True)

        @pl.when(j + 3 < NCH)
        def _():
            pltpu.async_copy(src_hbm.at[gidx_v.at[j + 3]], rows1_v, gsem1)

    pltpu.make_async_copy(src_hbm.at[gidx_v.at[NCH - 1]], rows0_v,
                          gsem0).wait()
    pltpu.sync_copy(rows0_v, acc_sp.at[sidx_v.at[NCH - 1]], add=True)

    plsc.subcore_barrier()
    pltpu.sync_copy(acc_sp.at[pl.ds(sbase, RPT)],
                    acc_out.at[c, pl.ds(sbase, RPT)])


def _dot(a, b):
    return lax.dot_general(a, b, (((1,), (0,)), ((), ())),
                           preferred_element_type=jnp.float32,
                           precision=lax.Precision.HIGHEST)


def _xform_body(x_ref, w_ref, o_ref):
    o_ref[...] = _dot(x_ref[...], w_ref[...])


def _mid_body(p_ref, c_ref, o_ref):
    psum = p_ref[0] + p_ref[1]
    cnt = c_ref[0, :, 0:1] + c_ref[1, :, 0:1]
    inv = jnp.where(cnt > 0.0, 1.0 / cnt, 0.0)
    o_ref[...] = psum * inv


def _fin_body(p_ref, c_ref, bhg_ref, w1_ref, wfc_ref, bfc_ref, o_ref):
    psum = p_ref[0] + p_ref[1]
    cnt = c_ref[0, :, 8:9] + c_ref[1, :, 8:9]
    inv = jnp.where(cnt > 0.0, 1.0 / cnt, 0.0)
    h = jnp.maximum(psum * inv + bhg_ref[...], 0.0)
    h2 = jnp.maximum(_dot(h, w1_ref[...]), 0.0)
    o_ref[...] = _dot(h2, wfc_ref[...]) + bfc_ref[0, 0]


def kernel(x, hyperedge_index, W_hg, b_hg, W1, a1, a2, W_fc, b_fc):
    ni = hyperedge_index[0]
    ei = hyperedge_index[1]
    fl = jnp.arange(NW * PT - M, dtype=jnp.int32)
    gpad = fl % N
    spad = N + fl % TRASH
    gA = jnp.concatenate([ni, gpad]).reshape(NW, NCH, CH)
    sA = jnp.concatenate([ei, spad]).reshape(NW, NCH, CH)
    gB = jnp.concatenate([ei, gpad]).reshape(NW, NCH, CH)
    sB = jnp.concatenate([ni, spad]).reshape(NW, NCH, CH)

    xt = pl.pallas_call(
        _xform_body,
        grid=(GRID,),
        in_specs=[pl.BlockSpec((RB, 128), lambda i: (i, 0)),
                  pl.BlockSpec((128, D), lambda i: (0, 0))],
        out_specs=pl.BlockSpec((RB, D), lambda i: (i, 0)),
        out_shape=jax.ShapeDtypeStruct((N, D), jnp.float32),
    )(x, W_hg)

    pe, ce = _sc_stage_a(xt, gA, sA, sB)

    out_e = pl.pallas_call(
        _mid_body,
        grid=(GRID,),
        in_specs=[pl.BlockSpec((NC, RB, D), lambda i: (0, i, 0)),
                  pl.BlockSpec((NC, RB, 16), lambda i: (0, i, 0))],
        out_specs=pl.BlockSpec((RB, D), lambda i: (i, 0)),
        out_shape=jax.ShapeDtypeStruct((N, D), jnp.float32),
    )(pe, ce)

    pn = _sc_stage_b(out_e, gB, sB)

    y = pl.pallas_call(
        _fin_body,
        grid=(GRID,),
        in_specs=[pl.BlockSpec((NC, RB, D), lambda i: (0, i, 0)),
                  pl.BlockSpec((NC, RB, 16), lambda i: (0, i, 0)),
                  pl.BlockSpec((1, D), lambda i: (0, 0)),
                  pl.BlockSpec((D, DA), lambda i: (0, 0)),
                  pl.BlockSpec((DA, 1), lambda i: (0, 0)),
                  pl.BlockSpec((1, 1), lambda i: (0, 0))],
        out_specs=pl.BlockSpec((RB, 1), lambda i: (i, 0)),
        out_shape=jax.ShapeDtypeStruct((N, 1), jnp.float32),
    )(pn, ce, b_hg.reshape(1, D), W1, W_fc.reshape(DA, 1),
      b_fc.reshape(1, 1))
    return y


# speedup vs baseline: 48.8420x; 1.0120x over previous
import functools

import jax
import jax.numpy as jnp
from jax import lax
from jax.experimental import pallas as pl
from jax.experimental.pallas import tpu as pltpu
from jax.experimental.pallas import tpu_sc as plsc

N = 10000
M = 320000
D = 32
DA = 64
NC = 2
NS = 16
NW = NC * NS
PER_TILE = M // NW
CH = 128
NCH = -(-PER_TILE // CH)
PT = NCH * CH
PADN = PT - PER_TILE
TRASH = 240
ROWS = N + TRASH
RPT = ROWS // NS
RB = 2000
GRID = N // RB

_mesh = plsc.VectorSubcoreMesh(core_axis_name="c", subcore_axis_name="s")


@functools.partial(
    pl.kernel,
    out_type=(
        jax.ShapeDtypeStruct((NC, N, D), jnp.float32),
        jax.ShapeDtypeStruct((NC, N, 16), jnp.float32),
    ),
    mesh=_mesh,
    scratch_types=[
        pltpu.VMEM((NCH, CH), jnp.int32),
        pltpu.VMEM((NCH, CH), jnp.int32),
        pltpu.VMEM((CH, D), jnp.float32),
        pltpu.VMEM((CH, D), jnp.float32),
        pltpu.SemaphoreType.DMA,
        pltpu.SemaphoreType.DMA,
        pltpu.VMEM((CH, 16), jnp.float32),
        pltpu.VMEM((CH, 16), jnp.float32),
        pltpu.VMEM((CH, D), jnp.float32),
        pltpu.VMEM((CH, 16), jnp.float32),
        pltpu.VMEM_SHARED((ROWS, D), jnp.float32),
        pltpu.VMEM_SHARED((ROWS, 16), jnp.float32),
    ],
    compiler_params=pltpu.CompilerParams(use_tc_tiling_on_sc=False),
)
def _sc_stage_a(src_hbm, gidx_hbm, sidx_hbm, acc_out, cnt_out,
                gidx_v, sidx_v, rows0_v, rows1_v, gsem0, gsem1,
                eones_v, nones_v, zrow_v, zcnt_v, acc_sp, cnt_sp):
    c = lax.axis_index("c")
    s = lax.axis_index("s")
    wid = c * NS + s

    zf = jnp.zeros((16,), jnp.float32)
    lane = lax.iota(jnp.int32, 16)
    e_pat = jnp.where(lane < 8, 1.0, 0.0).astype(jnp.float32)
    n_pat = jnp.where(lane < 8, 0.0, 1.0).astype(jnp.float32)

    @pl.loop(0, CH)
    def _(r):
        zrow_v[r, pl.ds(0, 16)] = zf
        zrow_v[r, pl.ds(16, 16)] = zf
        zcnt_v[r, pl.ds(0, 16)] = zf
        eones_v[r, pl.ds(0, 16)] = e_pat
        nones_v[r, pl.ds(0, 16)] = n_pat

    @pl.loop(0, RPT // CH)
    def _(k):
        base = s * RPT + k * CH
        pltpu.sync_copy(zrow_v, acc_sp.at[pl.ds(base, CH)])
        pltpu.sync_copy(zcnt_v, cnt_sp.at[pl.ds(base, CH)])

    sbase = pl.multiple_of(jnp.minimum(s * RPT, N - RPT), 16)
    pltpu.sync_copy(gidx_hbm.at[wid], gidx_v)
    pltpu.sync_copy(sidx_hbm.at[wid], sidx_v)
    pltpu.async_copy(src_hbm.at[gidx_v.at[0]], rows0_v, gsem0)
    pltpu.async_copy(src_hbm.at[gidx_v.at[1]], rows1_v, gsem1)
    plsc.subcore_barrier()

    @pl.loop(0, NCH - 1, step=2)
    def _(j):
        pltpu.make_async_copy(src_hbm.at[gidx_v.at[j]], rows0_v, gsem0).wait()
        pltpu.sync_copy(rows0_v, acc_sp.at[sidx_v.at[j]], add=True)
        pltpu.sync_copy(eones_v, cnt_sp.at[sidx_v.at[j]], add=True)
        pltpu.sync_copy(nones_v, cnt_sp.at[gidx_v.at[j]], add=True)

        @pl.when(j + 2 < NCH)
        def _():
            pltpu.async_copy(src_hbm.at[gidx_v.at[j + 2]], rows0_v, gsem0)

        pltpu.make_async_copy(src_hbm.at[gidx_v.at[j + 1]], rows1_v,
                              gsem1).wait()
        pltpu.sync_copy(rows1_v, acc_sp.at[sidx_v.at[j + 1]], add=True)
        pltpu.sync_copy(eones_v, cnt_sp.at[sidx_v.at[j + 1]], add=True)
        pltpu.sync_copy(nones_v, cnt_sp.at[gidx_v.at[j + 1]], add=True)

        @pl.when(j + 3 < NCH)
        def _():
            pltpu.async_copy(src_hbm.at[gidx_v.at[j + 3]], rows1_v, gsem1)

    pltpu.make_async_copy(src_hbm.at[gidx_v.at[NCH - 1]], rows0_v,
                          gsem0).wait()
    pltpu.sync_copy(rows0_v, acc_sp.at[sidx_v.at[NCH - 1]], add=True)
    pltpu.sync_copy(eones_v, cnt_sp.at[sidx_v.at[NCH - 1]], add=True)
    pltpu.sync_copy(nones_v, cnt_sp.at[gidx_v.at[NCH - 1]], add=True)

    plsc.subcore_barrier()
    pltpu.sync_copy(acc_sp.at[pl.ds(sbase, RPT)],
                    acc_out.at[c, pl.ds(sbase, RPT)])
    pltpu.sync_copy(cnt_sp.at[pl.ds(sbase, RPT)],
                    cnt_out.at[c, pl.ds(sbase, RPT)])


@functools.partial(
    pl.kernel,
    out_type=jax.ShapeDtypeStruct((NC, N, D), jnp.float32),
    mesh=_mesh,
    scratch_types=[
        pltpu.VMEM((NCH, CH), jnp.int32),
        pltpu.VMEM((NCH, CH), jnp.int32),
        pltpu.VMEM((CH, D), jnp.float32),
        pltpu.VMEM((CH, D), jnp.float32),
        pltpu.SemaphoreType.DMA,
        pltpu.SemaphoreType.DMA,
        pltpu.VMEM((CH, D), jnp.float32),
        pltpu.VMEM_SHARED((ROWS, D), jnp.float32),
    ],
    compiler_params=pltpu.CompilerParams(use_tc_tiling_on_sc=False),
)
def _sc_stage_b(src_hbm, gidx_hbm, sidx_hbm, acc_out,
                gidx_v, sidx_v, rows0_v, rows1_v, gsem0, gsem1,
                zrow_v, acc_sp):
    c = lax.axis_index("c")
    s = lax.axis_index("s")
    wid = c * NS + s

    zf = jnp.zeros((16,), jnp.float32)

    @pl.loop(0, CH)
    def _(r):
        zrow_v[r, pl.ds(0, 16)] = zf
        zrow_v[r, pl.ds(16, 16)] = zf

    @pl.loop(0, RPT // CH)
    def _(k):
        pltpu.sync_copy(zrow_v, acc_sp.at[pl.ds(s * RPT + k * CH, CH)])

    sbase = pl.multiple_of(jnp.minimum(s * RPT, N - RPT), 16)
    pltpu.sync_copy(gidx_hbm.at[wid], gidx_v)
    pltpu.sync_copy(sidx_hbm.at[wid], sidx_v)
    pltpu.async_copy(src_hbm.at[gidx_v.at[0]], rows0_v, gsem0)
    pltpu.async_copy(src_hbm.at[gidx_v.at[1]], rows1_v, gsem1)
    plsc.subcore_barrier()

    @pl.loop(0, NCH - 1, step=2)
    def _(j):
        pltpu.make_async_copy(src_hbm.at[gidx_v.at[j]], rows0_v, gsem0).wait()
        pltpu.sync_copy(rows0_v, acc_sp.at[sidx_v.at[j]], add=True)

        @pl.when(j + 2 < NCH)
        def _():
            pltpu.async_copy(src_hbm.at[gidx_v.at[j + 2]], rows0_v, gsem0)

        pltpu.make_async_copy(src_hbm.at[gidx_v.at[j + 1]], rows1_v,
                              gsem1).wait()
        pltpu.sync_copy(rows1_v, acc_sp.at[sidx_v.at[j + 1]], add=True)

        @pl.when(j + 3 < NCH)
        def _():
            pltpu.async_copy(src_hbm.at[gidx_v.at[j + 3]], rows1_v, gsem1)

    pltpu.make_async_copy(src_hbm.at[gidx_v.at[NCH - 1]], rows0_v,
                          gsem0).wait()
    pltpu.sync_copy(rows0_v, acc_sp.at[sidx_v.at[NCH - 1]], add=True)

    plsc.subcore_barrier()
    pltpu.sync_copy(acc_sp.at[pl.ds(sbase, RPT)],
                    acc_out.at[c, pl.ds(sbase, RPT)])


def _dot(a, b):
    return lax.dot_general(a, b, (((1,), (0,)), ((), ())),
                           preferred_element_type=jnp.float32,
                           precision=lax.Precision.HIGHEST)


def _xform_body(x_ref, w_ref, o_ref):
    o_ref[...] = _dot(x_ref[...], w_ref[...])


def _mid_body(p_ref, c_ref, o_ref):
    psum = p_ref[0] + p_ref[1]
    cnt = c_ref[0, :, 0:1] + c_ref[1, :, 0:1]
    inv = jnp.where(cnt > 0.0, 1.0 / cnt, 0.0)
    o_ref[...] = psum * inv


def _fin_body(p_ref, c_ref, bhg_ref, w1_ref, wfc_ref, bfc_ref, o_ref):
    psum = p_ref[0] + p_ref[1]
    cnt = c_ref[0, :, 8:9] + c_ref[1, :, 8:9]
    inv = jnp.where(cnt > 0.0, 1.0 / cnt, 0.0)
    h = jnp.maximum(psum * inv + bhg_ref[...], 0.0)
    h2 = jnp.maximum(_dot(h, w1_ref[...]), 0.0)
    o_ref[...] = _dot(h2, wfc_ref[...]) + bfc_ref[0, 0]


def kernel(x, hyperedge_index, W_hg, b_hg, W1, a1, a2, W_fc, b_fc):
    pad = N + jnp.arange(NW * PT - M, dtype=jnp.int32) % TRASH
    pN = jnp.concatenate([hyperedge_index[0], pad]).reshape(NW, NCH, CH)
    pE = jnp.concatenate([hyperedge_index[1], pad]).reshape(NW, NCH, CH)

    xt = pl.pallas_call(
        _xform_body,
        grid=(GRID,),
        in_specs=[pl.BlockSpec((ROWS // GRID, 128), lambda i: (i, 0)),
                  pl.BlockSpec((128, D), lambda i: (0, 0))],
        out_specs=pl.BlockSpec((ROWS // GRID, D), lambda i: (i, 0)),
        out_shape=jax.ShapeDtypeStruct((ROWS, D), jnp.float32),
    )(x, W_hg)

    pe, ce = _sc_stage_a(xt, pN, pE)

    out_e = pl.pallas_call(
        _mid_body,
        grid=(GRID,),
        in_specs=[pl.BlockSpec((NC, ROWS // GRID, D), lambda i: (0, i, 0)),
                  pl.BlockSpec((NC, ROWS // GRID, 16), lambda i: (0, i, 0))],
        out_specs=pl.BlockSpec((ROWS // GRID, D), lambda i: (i, 0)),
        out_shape=jax.ShapeDtypeStruct((ROWS, D), jnp.float32),
    )(pe, ce)

    pn = _sc_stage_b(out_e, pE, pN)

    y = pl.pallas_call(
        _fin_body,
        grid=(GRID,),
        in_specs=[pl.BlockSpec((NC, RB, D), lambda i: (0, i, 0)),
                  pl.BlockSpec((NC, RB, 16), lambda i: (0, i, 0)),
                  pl.BlockSpec((1, D), lambda i: (0, 0)),
                  pl.BlockSpec((D, DA), lambda i: (0, 0)),
                  pl.BlockSpec((DA, 1), lambda i: (0, 0)),
                  pl.BlockSpec((1, 1), lambda i: (0, 0))],
        out_specs=pl.BlockSpec((RB, 1), lambda i: (i, 0)),
        out_shape=jax.ShapeDtypeStruct((N, 1), jnp.float32),
    )(pn, ce, b_hg.reshape(1, D), W1, W_fc.reshape(DA, 1),
      b_fc.reshape(1, 1))
    return y


# speedup vs baseline: 49.9397x; 1.0225x over previous
import functools

import jax
import jax.numpy as jnp
from jax import lax
from jax.experimental import pallas as pl
from jax.experimental.pallas import tpu as pltpu
from jax.experimental.pallas import tpu_sc as plsc

N = 10000
M = 320000
D = 32
DA = 64
NC = 2
NS = 16
NW = NC * NS
PER_TILE = M // NW
CH = 128
NCH = -(-PER_TILE // CH)
PT = NCH * CH
PADN = PT - PER_TILE
TRASH = 240
ROWS = N + TRASH
RPT = ROWS // NS
RB = 2000
GRID = N // RB

_mesh = plsc.VectorSubcoreMesh(core_axis_name="c", subcore_axis_name="s")


@functools.partial(
    pl.kernel,
    out_type=(
        jax.ShapeDtypeStruct((NC, N, D), jnp.float32),
        jax.ShapeDtypeStruct((NC, N, 16), jnp.float32),
    ),
    mesh=_mesh,
    scratch_types=[
        pltpu.VMEM((NCH, CH), jnp.int32),
        pltpu.VMEM((NCH, CH), jnp.int32),
        pltpu.VMEM((CH, D), jnp.float32),
        pltpu.VMEM((CH, D), jnp.float32),
        pltpu.SemaphoreType.DMA,
        pltpu.SemaphoreType.DMA,
        pltpu.SemaphoreType.DMA,
        pltpu.SemaphoreType.DMA,
        pltpu.VMEM((CH, 16), jnp.float32),
        pltpu.VMEM((CH, 16), jnp.float32),
        pltpu.VMEM((CH, D), jnp.float32),
        pltpu.VMEM((CH, 16), jnp.float32),
        pltpu.VMEM_SHARED((ROWS, D), jnp.float32),
        pltpu.VMEM_SHARED((ROWS, 16), jnp.float32),
    ],
    compiler_params=pltpu.CompilerParams(use_tc_tiling_on_sc=False),
)
def _sc_stage_a(src_hbm, gidx_hbm, sidx_hbm, acc_out, cnt_out,
                gidx_v, sidx_v, rows0_v, rows1_v, gsem0, gsem1,
                csemE, csemN, eones_v, nones_v, zrow_v, zcnt_v,
                acc_sp, cnt_sp):
    c = lax.axis_index("c")
    s = lax.axis_index("s")
    wid = c * NS + s

    zf = jnp.zeros((16,), jnp.float32)
    lane = lax.iota(jnp.int32, 16)
    e_pat = jnp.where(lane < 8, 1.0, 0.0).astype(jnp.float32)
    n_pat = jnp.where(lane < 8, 0.0, 1.0).astype(jnp.float32)

    @pl.loop(0, CH)
    def _(r):
        zrow_v[r, pl.ds(0, 16)] = zf
        zrow_v[r, pl.ds(16, 16)] = zf
        zcnt_v[r, pl.ds(0, 16)] = zf
        eones_v[r, pl.ds(0, 16)] = e_pat
        nones_v[r, pl.ds(0, 16)] = n_pat

    @pl.loop(0, RPT // CH)
    def _(k):
        base = s * RPT + k * CH
        pltpu.sync_copy(zrow_v, acc_sp.at[pl.ds(base, CH)])
        pltpu.sync_copy(zcnt_v, cnt_sp.at[pl.ds(base, CH)])

    sbase = pl.multiple_of(jnp.minimum(s * RPT, N - RPT), 16)
    pltpu.sync_copy(gidx_hbm.at[wid], gidx_v)
    pltpu.sync_copy(sidx_hbm.at[wid], sidx_v)
    pltpu.async_copy(src_hbm.at[gidx_v.at[0]], rows0_v, gsem0)
    pltpu.async_copy(src_hbm.at[gidx_v.at[1]], rows1_v, gsem1)
    plsc.subcore_barrier()

    @pl.loop(0, NCH - 1, step=2)
    def _(j):
        pltpu.make_async_copy(src_hbm.at[gidx_v.at[j]], rows0_v, gsem0).wait()
        pltpu.sync_copy(rows0_v, acc_sp.at[sidx_v.at[j]], add=True)
        pltpu.async_copy(eones_v, cnt_sp.at[sidx_v.at[j]], csemE, add=True)
        pltpu.async_copy(nones_v, cnt_sp.at[gidx_v.at[j]], csemN, add=True)

        @pl.when(j + 2 < NCH)
        def _():
            pltpu.async_copy(src_hbm.at[gidx_v.at[j + 2]], rows0_v, gsem0)

        pltpu.make_async_copy(src_hbm.at[gidx_v.at[j + 1]], rows1_v,
                              gsem1).wait()
        pltpu.sync_copy(rows1_v, acc_sp.at[sidx_v.at[j + 1]], add=True)
        pltpu.async_copy(eones_v, cnt_sp.at[sidx_v.at[j + 1]], csemE,
                         add=True)
        pltpu.async_copy(nones_v, cnt_sp.at[gidx_v.at[j + 1]], csemN,
                         add=True)

        @pl.when(j + 3 < NCH)
        def _():
            pltpu.async_copy(src_hbm.at[gidx_v.at[j + 3]], rows1_v, gsem1)

    pltpu.make_async_copy(src_hbm.at[gidx_v.at[NCH - 1]], rows0_v,
                          gsem0).wait()
    pltpu.sync_copy(rows0_v, acc_sp.at[sidx_v.at[NCH - 1]], add=True)
    pltpu.async_copy(eones_v, cnt_sp.at[sidx_v.at[NCH - 1]], csemE, add=True)
    pltpu.async_copy(nones_v, cnt_sp.at[gidx_v.at[NCH - 1]], csemN, add=True)

    @pl.loop(0, NCH)
    def _(j):
        pltpu.make_async_copy(eones_v, cnt_sp.at[sidx_v.at[0]],
                              csemE).wait()
        pltpu.make_async_copy(nones_v, cnt_sp.at[gidx_v.at[0]],
                              csemN).wait()

    plsc.subcore_barrier()
    pltpu.sync_copy(acc_sp.at[pl.ds(sbase, RPT)],
                    acc_out.at[c, pl.ds(sbase, RPT)])
    pltpu.sync_copy(cnt_sp.at[pl.ds(sbase, RPT)],
                    cnt_out.at[c, pl.ds(sbase, RPT)])


@functools.partial(
    pl.kernel,
    out_type=jax.ShapeDtypeStruct((NC, N, D), jnp.float32),
    mesh=_mesh,
    scratch_types=[
        pltpu.VMEM((NCH, CH), jnp.int32),
        pltpu.VMEM((NCH, CH), jnp.int32),
        pltpu.VMEM((CH, D), jnp.float32),
        pltpu.VMEM((CH, D), jnp.float32),
        pltpu.SemaphoreType.DMA,
        pltpu.SemaphoreType.DMA,
        pltpu.VMEM((CH, D), jnp.float32),
        pltpu.VMEM_SHARED((ROWS, D), jnp.float32),
    ],
    compiler_params=pltpu.CompilerParams(use_tc_tiling_on_sc=False),
)
def _sc_stage_b(src_hbm, gidx_hbm, sidx_hbm, acc_out,
                gidx_v, sidx_v, rows0_v, rows1_v, gsem0, gsem1,
                zrow_v, acc_sp):
    c = lax.axis_index("c")
    s = lax.axis_index("s")
    wid = c * NS + s

    zf = jnp.zeros((16,), jnp.float32)

    @pl.loop(0, CH)
    def _(r):
        zrow_v[r, pl.ds(0, 16)] = zf
        zrow_v[r, pl.ds(16, 16)] = zf

    @pl.loop(0, RPT // CH)
    def _(k):
        pltpu.sync_copy(zrow_v, acc_sp.at[pl.ds(s * RPT + k * CH, CH)])

    sbase = pl.multiple_of(jnp.minimum(s * RPT, N - RPT), 16)
    pltpu.sync_copy(gidx_hbm.at[wid], gidx_v)
    pltpu.sync_copy(sidx_hbm.at[wid], sidx_v)
    pltpu.async_copy(src_hbm.at[gidx_v.at[0]], rows0_v, gsem0)
    pltpu.async_copy(src_hbm.at[gidx_v.at[1]], rows1_v, gsem1)
    plsc.subcore_barrier()

    @pl.loop(0, NCH - 1, step=2)
    def _(j):
        pltpu.make_async_copy(src_hbm.at[gidx_v.at[j]], rows0_v, gsem0).wait()
        pltpu.sync_copy(rows0_v, acc_sp.at[sidx_v.at[j]], add=True)

        @pl.when(j + 2 < NCH)
        def _():
            pltpu.async_copy(src_hbm.at[gidx_v.at[j + 2]], rows0_v, gsem0)

        pltpu.make_async_copy(src_hbm.at[gidx_v.at[j + 1]], rows1_v,
                              gsem1).wait()
        pltpu.sync_copy(rows1_v, acc_sp.at[sidx_v.at[j + 1]], add=True)

        @pl.when(j + 3 < NCH)
        def _():
            pltpu.async_copy(src_hbm.at[gidx_v.at[j + 3]], rows1_v, gsem1)

    pltpu.make_async_copy(src_hbm.at[gidx_v.at[NCH - 1]], rows0_v,
                          gsem0).wait()
    pltpu.sync_copy(rows0_v, acc_sp.at[sidx_v.at[NCH - 1]], add=True)

    plsc.subcore_barrier()
    pltpu.sync_copy(acc_sp.at[pl.ds(sbase, RPT)],
                    acc_out.at[c, pl.ds(sbase, RPT)])


def _dot(a, b):
    return lax.dot_general(a, b, (((1,), (0,)), ((), ())),
                           preferred_element_type=jnp.float32,
                           precision=lax.Precision.HIGHEST)


def _xform_body(x_ref, w_ref, o_ref):
    o_ref[...] = _dot(x_ref[...], w_ref[...])


def _mid_body(p_ref, c_ref, o_ref):
    psum = p_ref[0] + p_ref[1]
    cnt = c_ref[0, :, 0:1] + c_ref[1, :, 0:1]
    inv = jnp.where(cnt > 0.0, 1.0 / cnt, 0.0)
    o_ref[...] = psum * inv


def _fin_body(p_ref, c_ref, bhg_ref, w1_ref, wfc_ref, bfc_ref, o_ref):
    psum = p_ref[0] + p_ref[1]
    cnt = c_ref[0, :, 8:9] + c_ref[1, :, 8:9]
    inv = jnp.where(cnt > 0.0, 1.0 / cnt, 0.0)
    h = jnp.maximum(psum * inv + bhg_ref[...], 0.0)
    h2 = jnp.maximum(_dot(h, w1_ref[...]), 0.0)
    o_ref[...] = _dot(h2, wfc_ref[...]) + bfc_ref[0, 0]


def kernel(x, hyperedge_index, W_hg, b_hg, W1, a1, a2, W_fc, b_fc):
    pad = N + jnp.arange(NW * PT - M, dtype=jnp.int32) % TRASH
    pN = jnp.concatenate([hyperedge_index[0], pad]).reshape(NW, NCH, CH)
    pE = jnp.concatenate([hyperedge_index[1], pad]).reshape(NW, NCH, CH)

    xt = pl.pallas_call(
        _xform_body,
        grid=(GRID,),
        in_specs=[pl.BlockSpec((ROWS // GRID, 128), lambda i: (i, 0)),
                  pl.BlockSpec((128, D), lambda i: (0, 0))],
        out_specs=pl.BlockSpec((ROWS // GRID, D), lambda i: (i, 0)),
        out_shape=jax.ShapeDtypeStruct((ROWS, D), jnp.float32),
    )(x, W_hg)

    pe, ce = _sc_stage_a(xt, pN, pE)

    out_e = pl.pallas_call(
        _mid_body,
        grid=(1,),
        in_specs=[pl.BlockSpec((NC, ROWS, D), lambda i: (0, 0, 0)),
                  pl.BlockSpec((NC, ROWS, 16), lambda i: (0, 0, 0))],
        out_specs=pl.BlockSpec((ROWS, D), lambda i: (0, 0)),
        out_shape=jax.ShapeDtypeStruct((ROWS, D), jnp.float32),
    )(pe, ce)

    pn = _sc_stage_b(out_e, pE, pN)

    y = pl.pallas_call(
        _fin_body,
        grid=(1,),
        in_specs=[pl.BlockSpec((NC, N, D), lambda i: (0, 0, 0)),
                  pl.BlockSpec((NC, N, 16), lambda i: (0, 0, 0)),
                  pl.BlockSpec((1, D), lambda i: (0, 0)),
                  pl.BlockSpec((D, DA), lambda i: (0, 0)),
                  pl.BlockSpec((DA, 1), lambda i: (0, 0)),
                  pl.BlockSpec((1, 1), lambda i: (0, 0))],
        out_specs=pl.BlockSpec((N, 1), lambda i: (0, 0)),
        out_shape=jax.ShapeDtypeStruct((N, 1), jnp.float32),
    )(pn, ce, b_hg.reshape(1, D), W1, W_fc.reshape(DA, 1),
      b_fc.reshape(1, 1))
    return y


# speedup vs baseline: 61.2560x; 1.2266x over previous
import functools

import jax
import jax.numpy as jnp
from jax import lax
from jax.experimental import pallas as pl
from jax.experimental.pallas import tpu as pltpu
from jax.experimental.pallas import tpu_sc as plsc

N = 10000
M = 320000
D = 32
DA = 64
NC = 2
NS = 16
NW = NC * NS
CH = 128
NCH = 80
PT = NCH * CH
NBUF = 4
TRASH = 240
ROWS = N + TRASH
RPT = ROWS // NS
RB = 2000
GRID = N // RB

_mesh = plsc.VectorSubcoreMesh(core_axis_name="c", subcore_axis_name="s")


@functools.partial(
    pl.kernel,
    out_type=(
        jax.ShapeDtypeStruct((NC, N, D), jnp.float32),
        jax.ShapeDtypeStruct((NC, N, 16), jnp.float32),
    ),
    mesh=_mesh,
    scratch_types=[
        pltpu.VMEM((NCH, CH), jnp.int32),
        pltpu.VMEM((NCH, CH), jnp.int32),
        [pltpu.VMEM((CH, D), jnp.float32)] * NBUF,
        [pltpu.SemaphoreType.DMA] * NBUF,
        pltpu.SemaphoreType.DMA,
        pltpu.SemaphoreType.DMA,
        pltpu.VMEM((CH, 16), jnp.float32),
        pltpu.VMEM((CH, 16), jnp.float32),
        pltpu.VMEM((CH, D), jnp.float32),
        pltpu.VMEM((CH, 16), jnp.float32),
        pltpu.VMEM_SHARED((ROWS, D), jnp.float32),
        pltpu.VMEM_SHARED((ROWS, 16), jnp.float32),
    ],
    compiler_params=pltpu.CompilerParams(use_tc_tiling_on_sc=False),
)
def _sc_stage_a(src_hbm, gidx_hbm, sidx_hbm, acc_out, cnt_out,
                gidx_v, sidx_v, rows_v, gsem, csemE, csemN,
                eones_v, nones_v, zrow_v, zcnt_v, acc_sp, cnt_sp):
    c = lax.axis_index("c")
    s = lax.axis_index("s")
    wid = c * NS + s

    zf = jnp.zeros((16,), jnp.float32)
    lane = lax.iota(jnp.int32, 16)
    e_pat = jnp.where(lane < 8, 1.0, 0.0).astype(jnp.float32)
    n_pat = jnp.where(lane < 8, 0.0, 1.0).astype(jnp.float32)

    @pl.loop(0, CH)
    def _(r):
        zrow_v[r, pl.ds(0, 16)] = zf
        zrow_v[r, pl.ds(16, 16)] = zf
        zcnt_v[r, pl.ds(0, 16)] = zf
        eones_v[r, pl.ds(0, 16)] = e_pat
        nones_v[r, pl.ds(0, 16)] = n_pat

    @pl.loop(0, RPT // CH)
    def _(k):
        base = s * RPT + k * CH
        pltpu.sync_copy(zrow_v, acc_sp.at[pl.ds(base, CH)])
        pltpu.sync_copy(zcnt_v, cnt_sp.at[pl.ds(base, CH)])

    sbase = pl.multiple_of(jnp.minimum(s * RPT, N - RPT), 16)
    pltpu.sync_copy(gidx_hbm.at[wid], gidx_v)
    pltpu.sync_copy(sidx_hbm.at[wid], sidx_v)
    for b in range(NBUF):
        pltpu.async_copy(src_hbm.at[gidx_v.at[b]], rows_v[b], gsem[b])
    plsc.subcore_barrier()

    @pl.loop(0, NCH, step=NBUF)
    def _(j):
        for b in range(NBUF):
            pltpu.make_async_copy(src_hbm.at[gidx_v.at[j + b]], rows_v[b],
                                  gsem[b]).wait()
            pltpu.sync_copy(rows_v[b], acc_sp.at[sidx_v.at[j + b]], add=True)
            pltpu.async_copy(eones_v, cnt_sp.at[sidx_v.at[j + b]], csemE,
                             add=True)
            pltpu.async_copy(nones_v, cnt_sp.at[gidx_v.at[j + b]], csemN,
                             add=True)

            @pl.when(j + NBUF + b < NCH)
            def _():
                pltpu.async_copy(src_hbm.at[gidx_v.at[j + NBUF + b]],
                                 rows_v[b], gsem[b])

    @pl.loop(0, NCH)
    def _(j):
        pltpu.make_async_copy(eones_v, cnt_sp.at[sidx_v.at[0]], csemE).wait()
        pltpu.make_async_copy(nones_v, cnt_sp.at[gidx_v.at[0]], csemN).wait()

    plsc.subcore_barrier()
    pltpu.sync_copy(acc_sp.at[pl.ds(sbase, RPT)],
                    acc_out.at[c, pl.ds(sbase, RPT)])
    pltpu.sync_copy(cnt_sp.at[pl.ds(sbase, RPT)],
                    cnt_out.at[c, pl.ds(sbase, RPT)])


@functools.partial(
    pl.kernel,
    out_type=jax.ShapeDtypeStruct((NC, N, D), jnp.float32),
    mesh=_mesh,
    scratch_types=[
        pltpu.VMEM((NCH, CH), jnp.int32),
        pltpu.VMEM((NCH, CH), jnp.int32),
        [pltpu.VMEM((CH, D), jnp.float32)] * NBUF,
        [pltpu.SemaphoreType.DMA] * NBUF,
        pltpu.VMEM((CH, D), jnp.float32),
        pltpu.VMEM_SHARED((ROWS, D), jnp.float32),
    ],
    compiler_params=pltpu.CompilerParams(use_tc_tiling_on_sc=False),
)
def _sc_stage_b(src_hbm, gidx_hbm, sidx_hbm, acc_out,
                gidx_v, sidx_v, rows_v, gsem, zrow_v, acc_sp):
    c = lax.axis_index("c")
    s = lax.axis_index("s")
    wid = c * NS + s

    zf = jnp.zeros((16,), jnp.float32)

    @pl.loop(0, CH)
    def _(r):
        zrow_v[r, pl.ds(0, 16)] = zf
        zrow_v[r, pl.ds(16, 16)] = zf

    @pl.loop(0, RPT // CH)
    def _(k):
        pltpu.sync_copy(zrow_v, acc_sp.at[pl.ds(s * RPT + k * CH, CH)])

    sbase = pl.multiple_of(jnp.minimum(s * RPT, N - RPT), 16)
    pltpu.sync_copy(gidx_hbm.at[wid], gidx_v)
    pltpu.sync_copy(sidx_hbm.at[wid], sidx_v)
    for b in range(NBUF):
        pltpu.async_copy(src_hbm.at[gidx_v.at[b]], rows_v[b], gsem[b])
    plsc.subcore_barrier()

    @pl.loop(0, NCH, step=NBUF)
    def _(j):
        for b in range(NBUF):
            pltpu.make_async_copy(src_hbm.at[gidx_v.at[j + b]], rows_v[b],
                                  gsem[b]).wait()
            pltpu.sync_copy(rows_v[b], acc_sp.at[sidx_v.at[j + b]], add=True)

            @pl.when(j + NBUF + b < NCH)
            def _():
                pltpu.async_copy(src_hbm.at[gidx_v.at[j + NBUF + b]],
                                 rows_v[b], gsem[b])

    plsc.subcore_barrier()
    pltpu.sync_copy(acc_sp.at[pl.ds(sbase, RPT)],
                    acc_out.at[c, pl.ds(sbase, RPT)])


def _dot(a, b):
    return lax.dot_general(a, b, (((1,), (0,)), ((), ())),
                           preferred_element_type=jnp.float32)


def _xform_body(x_ref, w_ref, o_ref):
    o_ref[...] = _dot(x_ref[...], w_ref[...])


def _mid_body(p_ref, c_ref, o_ref):
    psum = p_ref[0] + p_ref[1]
    cnt = c_ref[0, :, 0:1] + c_ref[1, :, 0:1]
    inv = jnp.where(cnt > 0.0, 1.0 / cnt, 0.0)
    o_ref[...] = psum * inv


def _fin_body(p_ref, c_ref, bhg_ref, w1_ref, wfc_ref, bfc_ref, o_ref):
    psum = p_ref[0] + p_ref[1]
    cnt = c_ref[0, :, 8:9] + c_ref[1, :, 8:9]
    inv = jnp.where(cnt > 0.0, 1.0 / cnt, 0.0)
    h = jnp.maximum(psum * inv + bhg_ref[...], 0.0)
    h2 = jnp.maximum(_dot(h, w1_ref[...]), 0.0)
    o_ref[...] = _dot(h2, wfc_ref[...]) + bfc_ref[0, 0]


def kernel(x, hyperedge_index, W_hg, b_hg, W1, a1, a2, W_fc, b_fc):
    pad = N + jnp.arange(NW * PT - M, dtype=jnp.int32) % TRASH
    pN = jnp.concatenate([hyperedge_index[0], pad]).reshape(NW, NCH, CH)
    pE = jnp.concatenate([hyperedge_index[1], pad]).reshape(NW, NCH, CH)

    xt = pl.pallas_call(
        _xform_body,
        grid=(GRID,),
        in_specs=[pl.BlockSpec((ROWS // GRID, 128), lambda i: (i, 0)),
                  pl.BlockSpec((128, D), lambda i: (0, 0))],
        out_specs=pl.BlockSpec((ROWS // GRID, D), lambda i: (i, 0)),
        out_shape=jax.ShapeDtypeStruct((ROWS, D), jnp.float32),
    )(x, W_hg)

    pe, ce = _sc_stage_a(xt, pN, pE)

    out_e = pl.pallas_call(
        _mid_body,
        grid=(1,),
        in_specs=[pl.BlockSpec((NC, ROWS, D), lambda i: (0, 0, 0)),
                  pl.BlockSpec((NC, ROWS, 16), lambda i: (0, 0, 0))],
        out_specs=pl.BlockSpec((ROWS, D), lambda i: (0, 0)),
        out_shape=jax.ShapeDtypeStruct((ROWS, D), jnp.float32),
    )(pe, ce)

    pn = _sc_stage_b(out_e, pE, pN)

    y = pl.pallas_call(
        _fin_body,
        grid=(GRID,),
        in_specs=[pl.BlockSpec((NC, RB, D), lambda i: (0, i, 0)),
                  pl.BlockSpec((NC, RB, 16), lambda i: (0, i, 0)),
                  pl.BlockSpec((1, D), lambda i: (0, 0)),
                  pl.BlockSpec((D, DA), lambda i: (0, 0)),
                  pl.BlockSpec((DA, 1), lambda i: (0, 0)),
                  pl.BlockSpec((1, 1), lambda i: (0, 0))],
        out_specs=pl.BlockSpec((RB, 1), lambda i: (i, 0)),
        out_shape=jax.ShapeDtypeStruct((N, 1), jnp.float32),
    )(pn, ce, b_hg.reshape(1, D), W1, W_fc.reshape(DA, 1),
      b_fc.reshape(1, 1))
    return y


# speedup vs baseline: 61.4585x; 1.0033x over previous
import functools

import jax
import jax.numpy as jnp
from jax import lax
from jax.experimental import pallas as pl
from jax.experimental.pallas import tpu as pltpu
from jax.experimental.pallas import tpu_sc as plsc

N = 10000
M = 320000
D = 32
DA = 64
NC = 2
NS = 16
NW = NC * NS
CH = 128
NCH = 80
PT = NCH * CH
NBUF = 8
TRASH = 240
ROWS = N + TRASH
RPT = ROWS // NS
RB = 2000
GRID = N // RB

_mesh = plsc.VectorSubcoreMesh(core_axis_name="c", subcore_axis_name="s")


@functools.partial(
    pl.kernel,
    out_type=(
        jax.ShapeDtypeStruct((NC, N, D), jnp.float32),
        jax.ShapeDtypeStruct((NC, N, 16), jnp.float32),
    ),
    mesh=_mesh,
    scratch_types=[
        pltpu.VMEM((NCH, CH), jnp.int32),
        pltpu.VMEM((NCH, CH), jnp.int32),
        [pltpu.VMEM((CH, D), jnp.float32)] * NBUF,
        [pltpu.SemaphoreType.DMA] * NBUF,
        pltpu.SemaphoreType.DMA,
        pltpu.SemaphoreType.DMA,
        pltpu.VMEM((CH, 16), jnp.float32),
        pltpu.VMEM((CH, 16), jnp.float32),
        pltpu.VMEM((CH, D), jnp.float32),
        pltpu.VMEM((CH, 16), jnp.float32),
        pltpu.VMEM_SHARED((ROWS, D), jnp.float32),
        pltpu.VMEM_SHARED((ROWS, 16), jnp.float32),
    ],
    compiler_params=pltpu.CompilerParams(use_tc_tiling_on_sc=False),
)
def _sc_stage_a(src_hbm, gidx_hbm, sidx_hbm, acc_out, cnt_out,
                gidx_v, sidx_v, rows_v, gsem, csemE, csemN,
                eones_v, nones_v, zrow_v, zcnt_v, acc_sp, cnt_sp):
    c = lax.axis_index("c")
    s = lax.axis_index("s")
    wid = c * NS + s

    zf = jnp.zeros((16,), jnp.float32)
    lane = lax.iota(jnp.int32, 16)
    e_pat = jnp.where(lane < 8, 1.0, 0.0).astype(jnp.float32)
    n_pat = jnp.where(lane < 8, 0.0, 1.0).astype(jnp.float32)

    @pl.loop(0, CH)
    def _(r):
        zrow_v[r, pl.ds(0, 16)] = zf
        zrow_v[r, pl.ds(16, 16)] = zf
        zcnt_v[r, pl.ds(0, 16)] = zf
        eones_v[r, pl.ds(0, 16)] = e_pat
        nones_v[r, pl.ds(0, 16)] = n_pat

    @pl.loop(0, RPT // CH)
    def _(k):
        base = s * RPT + k * CH
        pltpu.sync_copy(zrow_v, acc_sp.at[pl.ds(base, CH)])
        pltpu.sync_copy(zcnt_v, cnt_sp.at[pl.ds(base, CH)])

    sbase = pl.multiple_of(jnp.minimum(s * RPT, N - RPT), 16)
    pltpu.sync_copy(gidx_hbm.at[wid], gidx_v)
    pltpu.sync_copy(sidx_hbm.at[wid], sidx_v)
    for b in range(NBUF):
        pltpu.async_copy(src_hbm.at[gidx_v.at[b]], rows_v[b], gsem[b])
    plsc.subcore_barrier()

    @pl.loop(0, NCH, step=NBUF)
    def _(j):
        for b in range(NBUF):
            pltpu.make_async_copy(src_hbm.at[gidx_v.at[j + b]], rows_v[b],
                                  gsem[b]).wait()
            pltpu.sync_copy(rows_v[b], acc_sp.at[sidx_v.at[j + b]], add=True)
            pltpu.async_copy(eones_v, cnt_sp.at[sidx_v.at[j + b]], csemE,
                             add=True)
            pltpu.async_copy(nones_v, cnt_sp.at[gidx_v.at[j + b]], csemN,
                             add=True)

            @pl.when(j + NBUF + b < NCH)
            def _():
                pltpu.async_copy(src_hbm.at[gidx_v.at[j + NBUF + b]],
                                 rows_v[b], gsem[b])

    @pl.loop(0, NCH)
    def _(j):
        pltpu.make_async_copy(eones_v, cnt_sp.at[sidx_v.at[0]], csemE).wait()
        pltpu.make_async_copy(nones_v, cnt_sp.at[gidx_v.at[0]], csemN).wait()

    plsc.subcore_barrier()
    pltpu.sync_copy(acc_sp.at[pl.ds(sbase, RPT)],
                    acc_out.at[c, pl.ds(sbase, RPT)])
    pltpu.sync_copy(cnt_sp.at[pl.ds(sbase, RPT)],
                    cnt_out.at[c, pl.ds(sbase, RPT)])


@functools.partial(
    pl.kernel,
    out_type=jax.ShapeDtypeStruct((NC, N, D), jnp.float32),
    mesh=_mesh,
    scratch_types=[
        pltpu.VMEM((NCH, CH), jnp.int32),
        pltpu.VMEM((NCH, CH), jnp.int32),
        [pltpu.VMEM((CH, D), jnp.float32)] * NBUF,
        [pltpu.SemaphoreType.DMA] * NBUF,
        pltpu.VMEM((CH, D), jnp.float32),
        pltpu.VMEM_SHARED((ROWS, D), jnp.float32),
    ],
    compiler_params=pltpu.CompilerParams(use_tc_tiling_on_sc=False),
)
def _sc_stage_b(src_hbm, gidx_hbm, sidx_hbm, acc_out,
                gidx_v, sidx_v, rows_v, gsem, zrow_v, acc_sp):
    c = lax.axis_index("c")
    s = lax.axis_index("s")
    wid = c * NS + s

    zf = jnp.zeros((16,), jnp.float32)

    @pl.loop(0, CH)
    def _(r):
        zrow_v[r, pl.ds(0, 16)] = zf
        zrow_v[r, pl.ds(16, 16)] = zf

    @pl.loop(0, RPT // CH)
    def _(k):
        pltpu.sync_copy(zrow_v, acc_sp.at[pl.ds(s * RPT + k * CH, CH)])

    sbase = pl.multiple_of(jnp.minimum(s * RPT, N - RPT), 16)
    pltpu.sync_copy(gidx_hbm.at[wid], gidx_v)
    pltpu.sync_copy(sidx_hbm.at[wid], sidx_v)
    for b in range(NBUF):
        pltpu.async_copy(src_hbm.at[gidx_v.at[b]], rows_v[b], gsem[b])
    plsc.subcore_barrier()

    @pl.loop(0, NCH, step=NBUF)
    def _(j):
        for b in range(NBUF):
            pltpu.make_async_copy(src_hbm.at[gidx_v.at[j + b]], rows_v[b],
                                  gsem[b]).wait()
            pltpu.sync_copy(rows_v[b], acc_sp.at[sidx_v.at[j + b]], add=True)

            @pl.when(j + NBUF + b < NCH)
            def _():
                pltpu.async_copy(src_hbm.at[gidx_v.at[j + NBUF + b]],
                                 rows_v[b], gsem[b])

    plsc.subcore_barrier()
    pltpu.sync_copy(acc_sp.at[pl.ds(sbase, RPT)],
                    acc_out.at[c, pl.ds(sbase, RPT)])


def _dot(a, b):
    return lax.dot_general(a, b, (((1,), (0,)), ((), ())),
                           preferred_element_type=jnp.float32)


def _xform_body(x_ref, w_ref, o_ref):
    o_ref[...] = _dot(x_ref[...], w_ref[...])


def _mid_body(p_ref, c_ref, o_ref):
    psum = p_ref[0] + p_ref[1]
    cnt = c_ref[0, :, 0:1] + c_ref[1, :, 0:1]
    inv = jnp.where(cnt > 0.0, 1.0 / cnt, 0.0)
    o_ref[...] = psum * inv


def _fin_body(p_ref, c_ref, bhg_ref, w1_ref, wfc_ref, bfc_ref, o_ref):
    psum = p_ref[0] + p_ref[1]
    cnt = c_ref[0, :, 8:9] + c_ref[1, :, 8:9]
    inv = jnp.where(cnt > 0.0, 1.0 / cnt, 0.0)
    h = jnp.maximum(psum * inv + bhg_ref[...], 0.0)
    h2 = jnp.maximum(_dot(h, w1_ref[...]), 0.0)
    o_ref[...] = _dot(h2, wfc_ref[...]) + bfc_ref[0, 0]


def kernel(x, hyperedge_index, W_hg, b_hg, W1, a1, a2, W_fc, b_fc):
    pad = N + jnp.arange(NW * PT - M, dtype=jnp.int32) % TRASH
    pNE = jnp.concatenate([hyperedge_index,
                           jnp.broadcast_to(pad, (2, pad.shape[0]))],
                          axis=1).reshape(2, NW, NCH, CH)
    pN = pNE[0]
    pE = pNE[1]

    xt = pl.pallas_call(
        _xform_body,
        grid=(GRID,),
        in_specs=[pl.BlockSpec((ROWS // GRID, 128), lambda i: (i, 0)),
                  pl.BlockSpec((128, D), lambda i: (0, 0))],
        out_specs=pl.BlockSpec((ROWS // GRID, D), lambda i: (i, 0)),
        out_shape=jax.ShapeDtypeStruct((ROWS, D), jnp.float32),
    )(x, W_hg)

    pe, ce = _sc_stage_a(xt, pN, pE)

    out_e = pl.pallas_call(
        _mid_body,
        grid=(1,),
        in_specs=[pl.BlockSpec((NC, ROWS, D), lambda i: (0, 0, 0)),
                  pl.BlockSpec((NC, ROWS, 16), lambda i: (0, 0, 0))],
        out_specs=pl.BlockSpec((ROWS, D), lambda i: (0, 0)),
        out_shape=jax.ShapeDtypeStruct((ROWS, D), jnp.float32),
    )(pe, ce)

    pn = _sc_stage_b(out_e, pE, pN)

    y = pl.pallas_call(
        _fin_body,
        grid=(GRID,),
        in_specs=[pl.BlockSpec((NC, RB, D), lambda i: (0, i, 0)),
                  pl.BlockSpec((NC, RB, 16), lambda i: (0, i, 0)),
                  pl.BlockSpec((1, D), lambda i: (0, 0)),
                  pl.BlockSpec((D, DA), lambda i: (0, 0)),
                  pl.BlockSpec((DA, 1), lambda i: (0, 0)),
                  pl.BlockSpec((1, 1), lambda i: (0, 0))],
        out_specs=pl.BlockSpec((RB, 1), lambda i: (i, 0)),
        out_shape=jax.ShapeDtypeStruct((N, 1), jnp.float32),
    )(pn, ce, b_hg.reshape(1, D), W1, W_fc.reshape(DA, 1),
      b_fc.reshape(1, 1))
    return y
